# baseline (speedup 1.0000x reference)
import jax
import jax.numpy as jnp
from jax.experimental import pallas as pl

N = 10000
E = 320000
DIM = 64
D_OUT = 10
NUM_GRAPHS = 128
DELTA = 2.5749


def _pna_conv(x, src, dst, p):
    msgs = jnp.take(x, src, axis=0)
    counts = jax.ops.segment_sum(jnp.ones((msgs.shape[0],), x.dtype), dst, num_segments=N)
    sums = jax.ops.segment_sum(msgs, dst, num_segments=N)
    maxs = jax.ops.segment_max(msgs, dst, num_segments=N)
    maxs = jnp.where(jnp.isneginf(maxs), 0.0, maxs)
    cnt = jnp.maximum(counts, 1.0)[:, None]
    means = sums / cnt
    mean_sq = jax.ops.segment_sum(msgs * msgs, dst, num_segments=N) / cnt
    var = jax.nn.relu(mean_sq - means * means)
    aggrs = [sums, maxs, means, var]
    c = counts[:, None]
    c_safe = jnp.where(c > 0, c, 1.0)
    amp = [c_safe / DELTA * a for a in aggrs]
    att = [DELTA / c_safe * a for a in aggrs]
    comb = jnp.concatenate(aggrs + amp + att, axis=1)
    aggr_out = comb @ p["Wm"] + p["bm"]
    z = x + aggr_out
    return jax.nn.relu(z @ p["Wa"] + p["ba"]) @ p["Wb"] + p["bb"]


def _bn(h, g, b):
    mu = jnp.mean(h, axis=0)
    v = jnp.var(h, axis=0)
    return (h - mu) / jnp.sqrt(v + 1e-5) * g + b


def _tail_kernel(h_ref, batch_ref, w1_ref, b1_ref, w2_ref, b2_ref, out_ref):
    h = h_ref[...]
    batch = batch_ref[...]
    seg = jax.lax.broadcasted_iota(jnp.int32, (NUM_GRAPHS, N), 0)
    onehot = jnp.where(batch == seg, 1.0, 0.0)
    g = jnp.dot(onehot, h, preferred_element_type=jnp.float32)
    g = jax.nn.relu(g @ w1_ref[...] + b1_ref[...])
    g = g @ w2_ref[...] + b2_ref[...]
    m = jnp.max(g, axis=-1, keepdims=True)
    e = jnp.exp(g - m)
    out_ref[...] = g - m - jnp.log(jnp.sum(e, axis=-1, keepdims=True))


def kernel(x, edge_index, batch, params):
    src, dst = edge_index[0], edge_index[1]
    h = jax.nn.relu(_pna_conv(x, src, dst, params["conv1"]))
    h = _bn(h, params["bn1"]["g"], params["bn1"]["b"])
    h = jax.nn.relu(_pna_conv(h, src, dst, params["conv2"]))
    h = _bn(h, params["bn2"]["g"], params["bn2"]["b"])
    h = jax.nn.relu(_pna_conv(h, src, dst, params["conv3"]))
    h = _bn(h, params["bn3"]["g"], params["bn3"]["b"])
    out = pl.pallas_call(
        _tail_kernel,
        out_shape=jax.ShapeDtypeStruct((NUM_GRAPHS, D_OUT), jnp.float32),
    )(h, batch.reshape(1, N), params["fc1"]["W"], params["fc1"]["b"].reshape(1, DIM),
      params["fc2"]["W"], params["fc2"]["b"].reshape(1, D_OUT))
    return out


# speedup vs baseline: 2.2470x; 2.2470x over previous
import functools

import jax
import jax.numpy as jnp
from jax import lax
from jax.experimental import pallas as pl
from jax.experimental.pallas import tpu as pltpu
from jax.experimental.pallas import tpu_sc as plsc

N = 10000
E = 320000
D_IN = 128
DIM = 64
D_OUT = 10
NUM_GRAPHS = 128
DELTA = 2.5749

NW = 32
RNG = 313
RPW = 314
NPAD = NW * RPW
CH = 4000
NCH = E // CH
F = 2048
RING = 2 * F
CAP = 321536
Q = 32
CNTW = 336

_MESH = plsc.VectorSubcoreMesh(core_axis_name="c", subcore_axis_name="s")
_SC_PARAMS = pltpu.CompilerParams(needs_layout_passes=False,
                                  use_tc_tiling_on_sc=False)


def _worker_id():
    return lax.axis_index("s") * 2 + lax.axis_index("c")


def _phase0_body(src_hbm, dst_hbm, ls_hbm, ld_hbm, cnt_hbm,
                 srcc, dstc, stage_s, stage_d, cntv):
    w = _worker_id()
    lo = w * RNG

    def flush(fl):
        off = pl.multiple_of(lax.bitwise_and(fl, RING - 1), F)
        gl = pl.multiple_of(w * CAP + fl, F)
        pltpu.sync_copy(stage_s.at[pl.ds(off, F)], ls_hbm.at[pl.ds(gl, F)])
        pltpu.sync_copy(stage_d.at[pl.ds(off, F)], ld_hbm.at[pl.ds(gl, F)])

    def chunk_body(ci, carry):
        pltpu.sync_copy(src_hbm.at[pl.ds(ci * CH, CH)], srcc)
        pltpu.sync_copy(dst_hbm.at[pl.ds(ci * CH, CH)], dstc)

        def vreg_body(v, carry2):
            cur, flushed = carry2
            dv = dstc[pl.ds(v * 16, 16)]
            sv = srcc[pl.ds(v * 16, 16)]
            t = dv - lo
            m = (t >= 0) & (t < RNG)
            mi = jnp.where(m, 1, 0).astype(jnp.int32)
            incl = plsc.cumsum(mi)
            wp = lax.bitwise_and(incl + (cur - 1), RING - 1)
            plsc.store_scatter(stage_s, [wp], sv, mask=m)
            plsc.store_scatter(stage_d, [wp], t, mask=m)
            cur = cur + incl[15]
            do = (cur - flushed) >= F

            @pl.when(do)
            def _():
                flush(flushed)

            flushed = flushed + jnp.where(do, F, 0)
            return cur, flushed

        return lax.fori_loop(0, CH // 16, vreg_body, carry)

    cur, flushed = lax.fori_loop(0, NCH, chunk_body,
                                 (jnp.int32(0), jnp.int32(0)))

    pad = lax.rem(Q - lax.rem(cur, Q), Q)
    iota = lax.iota(jnp.int32, 16)
    for k in range(Q // 16):
        off = k * 16
        m = (off + iota) < pad
        wp = lax.bitwise_and(cur + off + iota, RING - 1)
        plsc.store_scatter(stage_s, [wp], jnp.zeros((16,), jnp.int32), mask=m)
        plsc.store_scatter(stage_d, [wp], jnp.full((16,), RNG, jnp.int32),
                           mask=m)
    cur = cur + pad

    def drain(i, fl):
        do = fl < cur

        @pl.when(do)
        def _():
            flush(fl)

        return fl + jnp.where(do, F, 0)

    lax.fori_loop(0, 2, drain, flushed)
    cntv[0, pl.ds(0, 16)] = jnp.full((16,), cur, jnp.int32)
    pltpu.sync_copy(cntv, cnt_hbm.at[w])


def _phase0(src, dst):
    out_type = (
        jax.ShapeDtypeStruct((NW * CAP,), jnp.int32),
        jax.ShapeDtypeStruct((NW * CAP,), jnp.int32),
        jax.ShapeDtypeStruct((NW, 1, 16), jnp.int32),
    )
    scratch = [
        pltpu.VMEM((CH,), jnp.int32),
        pltpu.VMEM((CH,), jnp.int32),
        pltpu.VMEM((RING,), jnp.int32),
        pltpu.VMEM((RING,), jnp.int32),
        pltpu.VMEM((1, 16), jnp.int32),
    ]
    return pl.kernel(_phase0_body, out_type=out_type, mesh=_MESH,
                     scratch_types=scratch,
                     compiler_params=_SC_PARAMS)(src, dst)


def _agg_body(d, x_hbm, ls_hbm, ld_hbm, cnt_hbm,
              sums_hbm, maxs_hbm, ssq_hbm, cnts_hbm,
              sums, maxs, ssq, rows, sidx, dstl, cntw, cnta, sem):
    ncol = d // 16
    w = _worker_id()
    zeros = jnp.zeros((16,), jnp.float32)
    ninf = jnp.full((16,), -jnp.inf, jnp.float32)

    izeros = jnp.zeros((16,), jnp.int32)
    lane0 = jnp.where(lax.iota(jnp.int32, 16) == 0, 1, 0)

    def init_body(r, _):
        for cc in range(ncol):
            sl = pl.ds(cc * 16, 16)
            sums[r, sl] = zeros
            maxs[r, sl] = ninf
            ssq[r, sl] = zeros
        return 0

    lax.fori_loop(0, RPW, init_body, 0)
    for r in range(CNTW // 16):
        cnta[0, pl.ds(r * 16, 16)] = izeros

    pltpu.sync_copy(cnt_hbm.at[w], cntw)
    n_edges = cntw[0, pl.ds(0, 16)][0]
    nchunks = lax.div(n_edges, jnp.int32(Q))

    def chunk(ci, _):
        base = pl.multiple_of(w * CAP + ci * Q, Q)
        pltpu.sync_copy(ls_hbm.at[pl.ds(base, Q)], sidx)
        pltpu.sync_copy(ld_hbm.at[pl.ds(base, Q)], dstl)
        pltpu.async_copy(x_hbm.at[sidx], rows, sem).wait()

        for g in range(Q // 16):
            dlv = dstl[pl.ds(g * 16, 16)]
            for e in range(16):
                dl = dlv[e]
                row = g * 16 + e
                for cc in range(ncol):
                    sl = pl.ds(cc * 16, 16)
                    r = rows[row, sl]
                    sums[dl, sl] = sums[dl, sl] + r
                    maxs[dl, sl] = jnp.maximum(maxs[dl, sl], r)
                    ssq[dl, sl] = ssq[dl, sl] + r * r
                cnta[0, pl.ds(dl, 16)] = cnta[0, pl.ds(dl, 16)] + lane0
        return 0

    lax.fori_loop(0, nchunks, chunk, 0)
    pltpu.sync_copy(sums, sums_hbm.at[w])
    pltpu.sync_copy(maxs, maxs_hbm.at[w])
    pltpu.sync_copy(ssq, ssq_hbm.at[w])
    pltpu.sync_copy(cnta, cnts_hbm.at[w])


def _agg(x, ls, ld, cnt, d):
    out_type = (
        jax.ShapeDtypeStruct((NW, RPW, d), jnp.float32),
        jax.ShapeDtypeStruct((NW, RPW, d), jnp.float32),
        jax.ShapeDtypeStruct((NW, RPW, d), jnp.float32),
        jax.ShapeDtypeStruct((NW, 1, CNTW), jnp.int32),
    )
    scratch = [
        pltpu.VMEM((RPW, d), jnp.float32),
        pltpu.VMEM((RPW, d), jnp.float32),
        pltpu.VMEM((RPW, d), jnp.float32),
        pltpu.VMEM((Q, d), jnp.float32),
        pltpu.VMEM((Q,), jnp.int32),
        pltpu.VMEM((Q,), jnp.int32),
        pltpu.VMEM((1, 16), jnp.int32),
        pltpu.VMEM((1, CNTW), jnp.int32),
        pltpu.SemaphoreType.DMA,
    ]
    return pl.kernel(functools.partial(_agg_body, d), out_type=out_type,
                     mesh=_MESH, scratch_types=scratch,
                     compiler_params=_SC_PARAMS)(x, ls, ld, cnt)


def _unpad_rows(a, d):
    return a[:, :RNG].reshape(NW * RNG, d)[:N]


_BLK = 1000


def _dense_a_body(x_ref, sums_ref, maxs_ref, ssq_ref, cnt_ref,
                  wm_ref, bm_ref,
                  wa_ref, ba_ref, wb_ref, bb_ref, out_ref):
    x = x_ref[...]
    sums = sums_ref[...]
    maxs = maxs_ref[...]
    ssq = ssq_ref[...]
    c = cnt_ref[...]
    cnt = jnp.maximum(c, 1.0)
    means = sums / cnt
    var = jax.nn.relu(ssq / cnt - means * means)
    maxs = jnp.where(c > 0.0, maxs, 0.0)
    csafe = jnp.where(c > 0.0, c, 1.0)
    aggrs = [sums, maxs, means, var]
    amp = [csafe / DELTA * a for a in aggrs]
    att = [DELTA / csafe * a for a in aggrs]
    comb = jnp.concatenate(aggrs + amp + att, axis=1)

    def mm(a, w):
        return jnp.dot(a, w, preferred_element_type=jnp.float32)

    z = x + mm(comb, wm_ref[...]) + bm_ref[...]
    h1 = jax.nn.relu(mm(z, wa_ref[...]) + ba_ref[...])
    out_ref[...] = jax.nn.relu(mm(h1, wb_ref[...]) + bb_ref[...])


def _dense_b_body(h_ref, g_ref, b_ref, out_ref):
    h = h_ref[...]
    mu = jnp.mean(h, axis=0, keepdims=True)
    v = jnp.mean(h * h, axis=0, keepdims=True) - mu * mu
    out_ref[...] = (h - mu) * lax.rsqrt(v + 1e-5) * g_ref[...] + b_ref[...]


def _dense(x, sums, maxs, ssq, counts, conv, bn, d):
    nblk = N // _BLK
    row_spec = pl.BlockSpec((_BLK, d), lambda i: (i, 0))
    full = lambda *shape: pl.BlockSpec(shape, lambda i: (0,) * len(shape))
    h = pl.pallas_call(
        _dense_a_body,
        grid=(nblk,),
        in_specs=[row_spec, row_spec, row_spec, row_spec,
                  pl.BlockSpec((_BLK, 1), lambda i: (i, 0)),
                  full(12 * d, d),
                  full(1, d), full(d, DIM), full(1, DIM),
                  full(DIM, DIM), full(1, DIM)],
        out_specs=pl.BlockSpec((_BLK, DIM), lambda i: (i, 0)),
        out_shape=jax.ShapeDtypeStruct((N, DIM), jnp.float32),
    )(x, sums, maxs, ssq, counts, conv["Wm"],
      conv["bm"].reshape(1, d), conv["Wa"], conv["ba"].reshape(1, DIM),
      conv["Wb"], conv["bb"].reshape(1, DIM))
    return pl.pallas_call(
        _dense_b_body,
        out_shape=jax.ShapeDtypeStruct((N, DIM), jnp.float32),
    )(h, bn["g"].reshape(1, DIM), bn["b"].reshape(1, DIM))


def _tail_body(h_ref, batch_ref, w1_ref, b1_ref, w2_ref, b2_ref, out_ref):
    h = h_ref[...]
    batch = batch_ref[...]
    seg = lax.broadcasted_iota(jnp.int32, (NUM_GRAPHS, N), 0)
    onehot = jnp.where(batch == seg, 1.0, 0.0)

    g = jnp.dot(onehot, h, preferred_element_type=jnp.float32,
                precision=lax.Precision.HIGHEST)
    g = jax.nn.relu(g @ w1_ref[...] + b1_ref[...])
    g = g @ w2_ref[...] + b2_ref[...]
    m = jnp.max(g, axis=-1, keepdims=True)
    e = jnp.exp(g - m)
    out_ref[...] = g - m - jnp.log(jnp.sum(e, axis=-1, keepdims=True))


def kernel(x, edge_index, batch, params):
    src, dst = edge_index[0], edge_index[1]
    ls, ld, cnt = _phase0(src, dst)

    h = x
    counts = None
    for li, d in enumerate((D_IN, DIM, DIM)):
        sums_p, maxs_p, ssq_p, cnts_p = _agg(h, ls, ld, cnt, d)
        if counts is None:
            counts = (cnts_p[:, 0, :RNG].reshape(-1)[:N]
                      .astype(jnp.float32).reshape(N, 1))
        sums = _unpad_rows(sums_p, d)
        maxs = _unpad_rows(maxs_p, d)
        ssq = _unpad_rows(ssq_p, d)
        name = f"conv{li + 1}"
        bn = params[f"bn{li + 1}"]
        h = _dense(h, sums, maxs, ssq, counts, params[name], bn, d)

    return pl.pallas_call(
        _tail_body,
        out_shape=jax.ShapeDtypeStruct((NUM_GRAPHS, D_OUT), jnp.float32),
    )(h, batch.reshape(1, N), params["fc1"]["W"],
      params["fc1"]["b"].reshape(1, DIM), params["fc2"]["W"],
      params["fc2"]["b"].reshape(1, D_OUT))


# speedup vs baseline: 3.4267x; 1.5250x over previous
import functools

import jax
import jax.numpy as jnp
from jax import lax
from jax.experimental import pallas as pl
from jax.experimental.pallas import tpu as pltpu
from jax.experimental.pallas import tpu_sc as plsc

N = 10000
E = 320000
D_IN = 128
DIM = 64
D_OUT = 10
NUM_GRAPHS = 128
DELTA = 2.5749

NW = 32
RNG = 313
RPW = 314
CH = 4000
NCH = E // CH
F = 2048
RING = 2 * F
CAP = 321536
Q = 128
CNTW = 336

_MESH = plsc.VectorSubcoreMesh(core_axis_name="c", subcore_axis_name="s")
_SC_PARAMS = pltpu.CompilerParams(needs_layout_passes=False,
                                  use_tc_tiling_on_sc=False)


def _worker_id():
    return lax.axis_index("s") * 2 + lax.axis_index("c")


def _phase0_body(src_hbm, dst_hbm, ls_hbm, ld_hbm, cnt_hbm,
                 srcc, dstc, stage_s, stage_d, cntv):
    w = _worker_id()
    lo = w * RNG

    def flush(fl):
        off = pl.multiple_of(lax.bitwise_and(fl, RING - 1), F)
        gl = pl.multiple_of(w * CAP + fl, F)
        pltpu.sync_copy(stage_s.at[pl.ds(off, F)], ls_hbm.at[pl.ds(gl, F)])
        pltpu.sync_copy(stage_d.at[pl.ds(off, F)], ld_hbm.at[pl.ds(gl, F)])

    def chunk_body(ci, carry):
        pltpu.sync_copy(src_hbm.at[pl.ds(ci * CH, CH)], srcc)
        pltpu.sync_copy(dst_hbm.at[pl.ds(ci * CH, CH)], dstc)

        def vreg_body(v, carry2):
            cur, flushed = carry2
            dv = dstc[pl.ds(v * 16, 16)]
            sv = srcc[pl.ds(v * 16, 16)]
            t = dv - lo
            m = (t >= 0) & (t < RNG)
            mi = jnp.where(m, 1, 0).astype(jnp.int32)
            incl = plsc.cumsum(mi)
            wp = lax.bitwise_and(incl + (cur - 1), RING - 1)
            plsc.store_scatter(stage_s, [wp], sv, mask=m)
            plsc.store_scatter(stage_d, [wp], t, mask=m)
            cur = cur + incl[15]
            do = (cur - flushed) >= F

            @pl.when(do)
            def _():
                flush(flushed)

            flushed = flushed + jnp.where(do, F, 0)
            return cur, flushed

        return lax.fori_loop(0, CH // 16, vreg_body, carry)

    cur, flushed = lax.fori_loop(0, NCH, chunk_body,
                                 (jnp.int32(0), jnp.int32(0)))

    pad = lax.rem(Q - lax.rem(cur, Q), Q)
    iota = lax.iota(jnp.int32, 16)
    for k in range(Q // 16):
        off = k * 16
        m = (off + iota) < pad
        wp = lax.bitwise_and(cur + off + iota, RING - 1)
        plsc.store_scatter(stage_s, [wp], jnp.zeros((16,), jnp.int32), mask=m)
        plsc.store_scatter(stage_d, [wp], jnp.full((16,), RNG, jnp.int32),
                           mask=m)
    cur = cur + pad

    def drain(i, fl):
        do = fl < cur

        @pl.when(do)
        def _():
            flush(fl)

        return fl + jnp.where(do, F, 0)

    lax.fori_loop(0, 2, drain, flushed)
    cntv[0, pl.ds(0, 16)] = jnp.full((16,), cur, jnp.int32)
    pltpu.sync_copy(cntv, cnt_hbm.at[w])


def _phase0(src, dst):
    out_type = (
        jax.ShapeDtypeStruct((NW * CAP,), jnp.int32),
        jax.ShapeDtypeStruct((NW * CAP,), jnp.int32),
        jax.ShapeDtypeStruct((NW, 1, 16), jnp.int32),
    )
    scratch = [
        pltpu.VMEM((CH,), jnp.int32),
        pltpu.VMEM((CH,), jnp.int32),
        pltpu.VMEM((RING,), jnp.int32),
        pltpu.VMEM((RING,), jnp.int32),
        pltpu.VMEM((1, 16), jnp.int32),
    ]
    return pl.kernel(_phase0_body, out_type=out_type, mesh=_MESH,
                     scratch_types=scratch,
                     compiler_params=_SC_PARAMS)(src, dst)


AGG_D = 64
NCOL = AGG_D // 16


def _agg_body(idx_mul, idx_add, with_counts,
              x_hbm, y_hbm, ls_hbm, ld_hbm, cnt_hbm,
              sums_hbm, maxs_hbm, ssq_hbm, cnts_hbm,
              sums_sh, ssq_sh, maxs, rows_x, rows_y, sidx, gidx, dstl, dstsh,
              cntw, cnta, sgx, sgy, ssa, ssb):
    w = _worker_id()
    s = lax.axis_index("s")
    shbase = s * RPW
    zeros = jnp.zeros((16,), jnp.float32)
    ninf = jnp.full((16,), -jnp.inf, jnp.float32)
    ones = jnp.full((16,), 1, jnp.int32)

    def init_body(r, _):
        for cc in range(NCOL):
            maxs[r, pl.ds(cc * 16, 16)] = ninf
        return 0

    lax.fori_loop(0, RPW, init_body, 0)
    for r in range(Q):
        for cc in range(NCOL):
            rows_x[r, pl.ds(cc * 16, 16)] = zeros
    nfull = RPW // Q
    rem = RPW - nfull * Q
    for k in range(nfull):
        pltpu.sync_copy(rows_x, sums_sh.at[pl.ds(shbase + k * Q, Q)])
        pltpu.sync_copy(rows_x, ssq_sh.at[pl.ds(shbase + k * Q, Q)])
    if rem:
        pltpu.sync_copy(rows_x.at[pl.ds(0, rem)],
                        sums_sh.at[pl.ds(shbase + nfull * Q, rem)])
        pltpu.sync_copy(rows_x.at[pl.ds(0, rem)],
                        ssq_sh.at[pl.ds(shbase + nfull * Q, rem)])
    if with_counts:
        izeros = jnp.zeros((16,), jnp.int32)
        for r in range(CNTW // 16):
            cnta[pl.ds(r * 16, 16)] = izeros

    pltpu.sync_copy(cnt_hbm.at[w], cntw)
    n_edges = cntw[0, pl.ds(0, 16)][0]
    nchunks = lax.div(n_edges, jnp.int32(Q))

    def chunk(ci, _):
        base = pl.multiple_of(w * CAP + ci * Q, Q)
        pltpu.sync_copy(ls_hbm.at[pl.ds(base, Q)], sidx)
        pltpu.sync_copy(ld_hbm.at[pl.ds(base, Q)], dstl)
        if idx_mul == 1 and idx_add == 0:
            gref = sidx
        else:
            for v in range(Q // 16):
                sl = pl.ds(v * 16, 16)
                gidx[sl] = sidx[sl] * idx_mul + idx_add
            gref = gidx
        cpx = pltpu.async_copy(x_hbm.at[gref], rows_x, sgx)
        cpy = pltpu.async_copy(y_hbm.at[gref], rows_y, sgy)
        for v in range(Q // 16):
            sl = pl.ds(v * 16, 16)
            dstsh[sl] = dstl[sl] + shbase
        cpx.wait()
        cpy.wait()
        ca = pltpu.async_copy(rows_x, sums_sh.at[dstsh], ssa, add=True)
        cb = pltpu.async_copy(rows_y, ssq_sh.at[dstsh], ssb, add=True)

        for g in range(Q // 16):
            dlv = dstl[pl.ds(g * 16, 16)]
            if with_counts:
                plsc.addupdate_scatter(cnta, [dlv], ones)
            for e in range(16):
                dl = dlv[e]
                row = g * 16 + e
                for cc in range(NCOL):
                    sl = pl.ds(cc * 16, 16)
                    maxs[dl, sl] = jnp.maximum(maxs[dl, sl], rows_x[row, sl])
        ca.wait()
        cb.wait()
        return 0

    lax.fori_loop(0, nchunks, chunk, 0)
    pltpu.sync_copy(sums_sh.at[pl.ds(shbase, RPW)], sums_hbm.at[w])
    pltpu.sync_copy(ssq_sh.at[pl.ds(shbase, RPW)], ssq_hbm.at[w])
    pltpu.sync_copy(maxs, maxs_hbm.at[w])
    if with_counts:
        pltpu.sync_copy(cnta, cnts_hbm.at[w])


def _agg(x, y, ls, ld, cnt, idx_mul, idx_add, with_counts):
    out_type = (
        jax.ShapeDtypeStruct((NW, RPW, AGG_D), jnp.float32),
        jax.ShapeDtypeStruct((NW, RPW, AGG_D), jnp.float32),
        jax.ShapeDtypeStruct((NW, RPW, AGG_D), jnp.float32),
        jax.ShapeDtypeStruct((NW, CNTW), jnp.int32),
    )
    scratch = [
        pltpu.VMEM_SHARED((16 * RPW, AGG_D), jnp.float32),
        pltpu.VMEM_SHARED((16 * RPW, AGG_D), jnp.float32),
        pltpu.VMEM((RPW, AGG_D), jnp.float32),
        pltpu.VMEM((Q, AGG_D), jnp.float32),
        pltpu.VMEM((Q, AGG_D), jnp.float32),
        pltpu.VMEM((Q,), jnp.int32),
        pltpu.VMEM((Q,), jnp.int32),
        pltpu.VMEM((Q,), jnp.int32),
        pltpu.VMEM((Q,), jnp.int32),
        pltpu.VMEM((1, 16), jnp.int32),
        pltpu.VMEM((CNTW,), jnp.int32),
        pltpu.SemaphoreType.DMA,
        pltpu.SemaphoreType.DMA,
        pltpu.SemaphoreType.DMA,
        pltpu.SemaphoreType.DMA,
    ]
    return pl.kernel(functools.partial(_agg_body, idx_mul, idx_add,
                                       with_counts),
                     out_type=out_type,
                     mesh=_MESH, scratch_types=scratch,
                     compiler_params=_SC_PARAMS)(x, y, ls, ld, cnt)


def _unpad_rows(a, d):
    return a[:, :RNG].reshape(NW * RNG, d)[:N]


_BLK = 1000


def _dense_a_body(x_ref, sums_ref, maxs_ref, ssq_ref, cnt_ref,
                  wm_ref, bm_ref,
                  wa_ref, ba_ref, wb_ref, bb_ref, out_ref):
    x = x_ref[...]
    sums = sums_ref[...]
    maxs = maxs_ref[...]
    ssq = ssq_ref[...]
    c = cnt_ref[...]
    cnt = jnp.maximum(c, 1.0)
    means = sums / cnt
    var = jax.nn.relu(ssq / cnt - means * means)
    maxs = jnp.where(c > 0.0, maxs, 0.0)
    csafe = jnp.where(c > 0.0, c, 1.0)
    aggrs = [sums, maxs, means, var]
    amp = [csafe / DELTA * a for a in aggrs]
    att = [DELTA / csafe * a for a in aggrs]
    comb = jnp.concatenate(aggrs + amp + att, axis=1)

    def mm(a, w):
        return jnp.dot(a, w, preferred_element_type=jnp.float32)

    z = x + mm(comb, wm_ref[...]) + bm_ref[...]
    h1 = jax.nn.relu(mm(z, wa_ref[...]) + ba_ref[...])
    out_ref[...] = jax.nn.relu(mm(h1, wb_ref[...]) + bb_ref[...])


def _dense_b_body(h_ref, g_ref, b_ref, out_ref, sq_ref):
    h = h_ref[...]
    mu = jnp.mean(h, axis=0, keepdims=True)
    v = jnp.mean(h * h, axis=0, keepdims=True) - mu * mu
    o = (h - mu) * lax.rsqrt(v + 1e-5) * g_ref[...] + b_ref[...]
    out_ref[...] = o
    sq_ref[...] = o * o


def _dense(x, sums, maxs, ssq, counts, conv, bn, d):
    nblk = N // _BLK
    row_spec = pl.BlockSpec((_BLK, d), lambda i: (i, 0))
    full = lambda *shape: pl.BlockSpec(shape, lambda i: (0,) * len(shape))
    h = pl.pallas_call(
        _dense_a_body,
        grid=(nblk,),
        in_specs=[row_spec, row_spec, row_spec, row_spec,
                  pl.BlockSpec((_BLK, 1), lambda i: (i, 0)),
                  full(12 * d, d),
                  full(1, d), full(d, DIM), full(1, DIM),
                  full(DIM, DIM), full(1, DIM)],
        out_specs=pl.BlockSpec((_BLK, DIM), lambda i: (i, 0)),
        out_shape=jax.ShapeDtypeStruct((N, DIM), jnp.float32),
    )(x, sums, maxs, ssq, counts, conv["Wm"],
      conv["bm"].reshape(1, d), conv["Wa"], conv["ba"].reshape(1, DIM),
      conv["Wb"], conv["bb"].reshape(1, DIM))
    return pl.pallas_call(
        _dense_b_body,
        out_shape=(jax.ShapeDtypeStruct((N, DIM), jnp.float32),
                   jax.ShapeDtypeStruct((N, DIM), jnp.float32)),
    )(h, bn["g"].reshape(1, DIM), bn["b"].reshape(1, DIM))


def _square_body(x_ref, y_ref):
    x = x_ref[...]
    y_ref[...] = x * x


def _square(x, d):
    row_spec = pl.BlockSpec((_BLK, d), lambda i: (i, 0))
    return pl.pallas_call(
        _square_body,
        grid=(N // _BLK,),
        in_specs=[row_spec],
        out_specs=row_spec,
        out_shape=jax.ShapeDtypeStruct((N, d), jnp.float32),
    )(x)


def _tail_body(h_ref, batch_ref, w1_ref, b1_ref, w2_ref, b2_ref, out_ref):
    h = h_ref[...]
    batch = batch_ref[...]
    seg = lax.broadcasted_iota(jnp.int32, (NUM_GRAPHS, N), 0)
    onehot = jnp.where(batch == seg, 1.0, 0.0)

    g = jnp.dot(onehot, h, preferred_element_type=jnp.float32,
                precision=lax.Precision.HIGHEST)
    g = jax.nn.relu(g @ w1_ref[...] + b1_ref[...])
    g = g @ w2_ref[...] + b2_ref[...]
    m = jnp.max(g, axis=-1, keepdims=True)
    e = jnp.exp(g - m)
    out_ref[...] = g - m - jnp.log(jnp.sum(e, axis=-1, keepdims=True))


def kernel(x, edge_index, batch, params):
    src, dst = edge_index[0], edge_index[1]
    ls, ld, cnt = _phase0(src, dst)

    h = x
    y = _square(x, D_IN)
    counts = None
    for li, d in enumerate((D_IN, DIM, DIM)):
        if d == D_IN:
            xv = h.reshape(2 * N, AGG_D)
            yv = y.reshape(2 * N, AGG_D)
            halves = []
            for hh in range(2):
                sums_p, maxs_p, ssq_p, cnts_p = _agg(
                    xv, yv, ls, ld, cnt, 2, hh, counts is None and hh == 0)
                if counts is None:
                    counts = (cnts_p[:, :RNG].reshape(-1)[:N]
                              .astype(jnp.float32).reshape(N, 1))
                halves.append((_unpad_rows(sums_p, AGG_D),
                               _unpad_rows(maxs_p, AGG_D),
                               _unpad_rows(ssq_p, AGG_D)))
            sums = jnp.stack([halves[0][0], halves[1][0]],
                             axis=1).reshape(N, d)
            maxs = jnp.stack([halves[0][1], halves[1][1]],
                             axis=1).reshape(N, d)
            ssq = jnp.stack([halves[0][2], halves[1][2]],
                            axis=1).reshape(N, d)
        else:
            sums_p, maxs_p, ssq_p, _ = _agg(h, y, ls, ld, cnt, 1, 0, False)
            sums = _unpad_rows(sums_p, AGG_D)
            maxs = _unpad_rows(maxs_p, AGG_D)
            ssq = _unpad_rows(ssq_p, AGG_D)
        name = f"conv{li + 1}"
        bn = params[f"bn{li + 1}"]
        h, y = _dense(h, sums, maxs, ssq, counts, params[name], bn, d)

    return pl.pallas_call(
        _tail_body,
        out_shape=jax.ShapeDtypeStruct((NUM_GRAPHS, D_OUT), jnp.float32),
    )(h, batch.reshape(1, N), params["fc1"]["W"],
      params["fc1"]["b"].reshape(1, DIM), params["fc2"]["W"],
      params["fc2"]["b"].reshape(1, D_OUT))
